# baseline (speedup 1.0000x reference)
import functools

import jax
import jax.numpy as jnp
from jax import lax
from jax.experimental import pallas as pl
from jax.experimental.pallas import tpu as pltpu
from jax.experimental.pallas import tpu_sc as plsc

B = 16384
D = 16
L = 16


@functools.lru_cache(maxsize=None)
def _build(nw: int):
    b_per_w = B // nw
    n_blocks = b_per_w // L
    mesh = plsc.VectorSubcoreMesh(core_axis_name="c", subcore_axis_name="s")

    @functools.partial(
        pl.kernel,
        mesh=mesh,
        out_type=jax.ShapeDtypeStruct((B,), jnp.float32),
        scratch_types=[
            pltpu.VMEM((b_per_w,), jnp.int32),
            pltpu.VMEM((b_per_w, D), jnp.float32),
            pltpu.VMEM((b_per_w, D), jnp.float32),
            pltpu.VMEM((b_per_w,), jnp.float32),
            pltpu.SemaphoreType.DMA,
        ],
        compiler_params=pltpu.CompilerParams(use_tc_tiling_on_sc=False),
    )
    def sc_fwd(x_hbm, a_hbm, m_hbm, out_hbm, idx_v, xs_v, ms_v, out_v, sem):
        nc = lax.axis_size("c")
        wid = lax.axis_index("s") * nc + lax.axis_index("c")
        base = wid * b_per_w

        pltpu.sync_copy(a_hbm.at[pl.ds(base, b_per_w)], idx_v)
        gather = pltpu.async_copy(m_hbm.at[idx_v], ms_v, sem)
        pltpu.sync_copy(x_hbm.at[pl.ds(base, b_per_w), :], xs_v)
        gather.wait()

        iota = lax.iota(jnp.int32, L)

        dnums = lax.GatherDimensionNumbers(
            offset_dims=(), collapsed_slice_dims=(0,), start_index_map=(0,))

        def permute(v, idx):
            return lax.gather(v, idx[:, None], dnums, slice_sizes=(1,),
                              mode=lax.GatherScatterMode.PROMISE_IN_BOUNDS)

        def lanesum(v):
            for sh in (1, 2, 4, 8):
                v = v + permute(v, iota ^ sh)
            return v

        def blk(b, carry):
            acc = jnp.zeros((L,), jnp.float32)
            for j in range(L):
                r = b * L + j
                p = xs_v[r, :] * ms_v[r, :]
                acc = jnp.where(iota == j, lanesum(p), acc)
            out_v[pl.ds(b * L, L)] = 1.0 / (1.0 + jnp.exp(-acc))
            return carry

        lax.fori_loop(0, n_blocks, blk, 0)
        pltpu.sync_copy(out_v, out_hbm.at[pl.ds(base, b_per_w)])

    return sc_fwd


def kernel(X, A, m):
    info = plsc.get_sparse_core_info()
    nw = info.num_cores * info.num_subcores
    return _build(nw)(X, A.astype(jnp.int32), m)
